# baseline (speedup 1.0000x reference)
import functools

import numpy as np
import jax
import jax.numpy as jnp
from jax import lax
from jax.experimental import pallas as pl
from jax.experimental.pallas import tpu as pltpu
from jax.experimental.pallas import tpu_sc as plsc

BOX = 10.0
ALPHA = 2.8
R_ONSET = 1.7
R_CUTOFF = 2.0
NSIDE = 5
NCELLS = NSIDE ** 3


def _cell_adjacency() -> np.ndarray:
    ids = np.arange(NCELLS)
    x, rem = divmod(ids, NSIDE * NSIDE)
    y, z = divmod(rem, NSIDE)
    def near(a, b):
        d = np.abs(a[:, None] - b[None, :])
        return np.minimum(d, NSIDE - d) <= 1
    adj = near(x, x) & near(y, y) & near(z, z)
    return adj.astype(np.float32)


_ADJ = _cell_adjacency()

_SC_CORES = 2
_SC_SUBCORES = 16


def _sc_permute_rows(table, idx):
    n, d = table.shape
    nw = _SC_CORES * _SC_SUBCORES
    b_per_w = n // nw
    mesh = plsc.VectorSubcoreMesh(core_axis_name="c", subcore_axis_name="s")

    @functools.partial(
        pl.kernel,
        out_type=jax.ShapeDtypeStruct((n, d), jnp.float32),
        mesh=mesh,
        scratch_types=[
            pltpu.VMEM((b_per_w,), jnp.int32),
            pltpu.VMEM((b_per_w, d), jnp.float32),
            pltpu.SemaphoreType.DMA,
        ],
    )
    def gather_kernel(table_hbm, idx_hbm, out_hbm, idx_v, rows_v, sem):
        wid = lax.axis_index("s") * _SC_CORES + lax.axis_index("c")
        base = wid * b_per_w
        pltpu.sync_copy(idx_hbm.at[pl.ds(base, b_per_w)], idx_v)
        pltpu.async_copy(table_hbm.at[idx_v], rows_v, sem).wait()
        pltpu.sync_copy(rows_v, out_hbm.at[pl.ds(base, b_per_w)])

    return gather_kernel(table, idx)


def _prologue_kernel(ct_ref, ctt_ref, cad_ref, rrow_ref, rcol_ref,
                     arow_ref, brow_ref, oht_ref, dcorr_ref):
    sig8 = rcol_ref[...] + rrow_ref[...]
    eps8 = cad_ref[...]
    e_sig = jnp.exp(ALPHA * sig8)
    a8 = eps8 * e_sig * e_sig
    b8 = 2.0 * eps8 * e_sig

    ct = ct_ref[...]
    n = ct.shape[0]
    mx = jnp.max(ct, axis=1, keepdims=True)
    iota = jax.lax.broadcasted_iota(jnp.int32, (n, 8), 1)
    idx = jnp.min(jnp.where(ct == mx, iota, 8), axis=1, keepdims=True)
    idx = jnp.where(jnp.sum(ct, axis=1, keepdims=True) > 0.0, idx, 0)
    oh = (iota == idx).astype(jnp.float32)
    arow_ref[...] = jnp.dot(oh, a8, preferred_element_type=jnp.float32)
    brow_ref[...] = jnp.dot(oh, b8, preferred_element_type=jnp.float32)

    ctt = ctt_ref[...]
    mxt = jnp.max(ctt, axis=0, keepdims=True)
    iota_t = jax.lax.broadcasted_iota(jnp.int32, (8, n), 0)
    idx_t = jnp.min(jnp.where(ctt == mxt, iota_t, 8), axis=0, keepdims=True)
    idx_t = jnp.where(jnp.sum(ctt, axis=0, keepdims=True) > 0.0, idx_t, 0)
    oht_ref[...] = (iota_t == idx_t).astype(jnp.float32)

    t1 = float(np.exp(-ALPHA))
    diag_iota = jax.lax.broadcasted_iota(jnp.int32, (8, 8), 0)
    eye8 = (diag_iota == jax.lax.broadcasted_iota(jnp.int32, (8, 8), 1))
    a_d = jnp.sum(jnp.where(eye8, a8, 0.0), axis=1, keepdims=True)
    b_d = jnp.sum(jnp.where(eye8, b8, 0.0), axis=1, keepdims=True)
    av = jnp.dot(oh, a_d, preferred_element_type=jnp.float32)
    bv = jnp.dot(oh, b_d, preferred_element_type=jnp.float32)
    dcorr_ref[...] = jnp.sum((av * t1 - bv) * t1).reshape(1, 1)


def _tile_kernel(chunks_ref, counts_ref, pr_ref, arow_ref, brow_ref,
                 pc_ref, oht_ref, out_ref, *, tr, cc):
    i = pl.program_id(0)

    pr = pr_ref[...]
    arow = arow_ref[...]
    brow = brow_ref[...]

    r_on2 = R_ONSET * R_ONSET
    r_c2 = R_CUTOFF * R_CUTOFF
    inv_den = 1.0 / (r_c2 - r_on2) ** 3

    def chunk_u(c):
        pc = pc_ref[c]
        oht = oht_ref[c]
        a_t = jnp.dot(arow, oht, preferred_element_type=jnp.float32)
        b_t = jnp.dot(brow, oht, preferred_element_type=jnp.float32)

        dr2 = jnp.zeros((tr, cc), jnp.float32)
        for kk in range(3):
            d = pr[:, kk:kk + 1] - pc[kk:kk + 1, :]
            d = d - BOX * jnp.round(d * (1.0 / BOX))
            dr2 = dr2 + d * d

        safe = jnp.where(dr2 > 0.0, dr2, 1.0)
        r = jnp.sqrt(safe)
        t = jnp.exp(-ALPHA * r)
        x = jnp.minimum(jnp.maximum(r * r, r_on2), r_c2)
        s = ((r_c2 - x) ** 2 * (r_c2 + 2.0 * x - 3.0 * r_on2)) * inv_den
        u = (a_t * t - b_t) * (t * s)
        return jnp.sum(u.reshape(tr // 8, 8, cc), axis=0)

    def body(k, acc):
        return acc + chunk_u(chunks_ref[i, k])

    acc = jax.lax.fori_loop(0, counts_ref[i], body,
                            jnp.zeros((8, cc), jnp.float32))
    total = 0.5 * (jnp.sum(chunk_u(i)) + 2.0 * jnp.sum(acc))

    @pl.when(i == 0)
    def _():
        out_ref[...] = jnp.zeros((1, 1), jnp.float32)

    out_ref[...] += total.reshape(1, 1)


def kernel(positions, celltype, cadherin, radius):
    n = positions.shape[0]
    tr, cc = 128, 128
    nr, nk = n // tr, n // cc

    ci = jnp.floor(positions * (NSIDE / BOX)).astype(jnp.int32)
    ci = jnp.clip(ci, 0, NSIDE - 1)
    cid = (ci[:, 0] * NSIDE + ci[:, 1]) * NSIDE + ci[:, 2]
    perm = jnp.argsort(cid).astype(jnp.int32)
    table = jnp.concatenate(
        [positions, celltype, cid[:, None].astype(jnp.float32),
         jnp.zeros((n, 116), jnp.float32)], axis=1)
    table_s = _sc_permute_rows(table, perm)
    pos_s = table_s[:, 0:3]
    ct_s = table_s[:, 3:11]
    cid_s = table_s[:, 11].astype(jnp.int32)

    oh_cell = (cid_s[:, None] == jnp.arange(NCELLS)[None, :])
    pres_r = jnp.any(oh_cell.reshape(nr, tr, NCELLS), axis=1)
    pres_c = jnp.any(oh_cell.reshape(nk, cc, NCELLS), axis=1)
    adj = jnp.asarray(_ADJ)
    reach = pres_r.astype(jnp.float32) @ adj
    active = (reach @ pres_c.T.astype(jnp.float32)) > 0.0
    rows = jnp.broadcast_to(jnp.arange(nr)[:, None], (nr, nk))
    cols = jnp.broadcast_to(jnp.arange(nk)[None, :], (nr, nk))
    upper = active & (cols > rows)
    counts = jnp.sum(upper, axis=1, dtype=jnp.int32)
    slot = jnp.cumsum(upper, axis=1) - 1
    chunk_list = jnp.zeros((nr, nk), jnp.int32).at[
        rows.ravel(),
        jnp.where(upper, slot, nk).ravel()
    ].set(cols.astype(jnp.int32).ravel(), mode='drop')

    pos_row = table_s[:, 0:8]
    pos_col3 = jnp.transpose(pos_row.T.reshape(8, nk, cc), (1, 0, 2))
    cad8 = jnp.reshape(cadherin, (8, 8))
    cad8 = 0.5 * (cad8 + cad8.T)
    r8_row = jnp.reshape(radius[:8, 0], (1, 8))
    r8_col = jnp.reshape(radius[:8, 0], (8, 1))

    arow, brow, oht, dcorr = pl.pallas_call(
        _prologue_kernel,
        out_shape=[
            jax.ShapeDtypeStruct((n, 8), jnp.float32),
            jax.ShapeDtypeStruct((n, 8), jnp.float32),
            jax.ShapeDtypeStruct((8, n), jnp.float32),
            jax.ShapeDtypeStruct((1, 1), jnp.float32),
        ],
    )(ct_s, ct_s.T, cad8, r8_row, r8_col)
    oht3 = jnp.transpose(oht.reshape(8, nk, cc), (1, 0, 2))

    out = pl.pallas_call(
        functools.partial(_tile_kernel, tr=tr, cc=cc),
        grid_spec=pltpu.PrefetchScalarGridSpec(
            num_scalar_prefetch=2,
            grid=(nr,),
            in_specs=[
                pl.BlockSpec((tr, 8), lambda i, *_: (i, 0)),
                pl.BlockSpec((tr, 8), lambda i, *_: (i, 0)),
                pl.BlockSpec((tr, 8), lambda i, *_: (i, 0)),
                pl.BlockSpec((nk, 8, cc), lambda i, *_: (0, 0, 0)),
                pl.BlockSpec((nk, 8, cc), lambda i, *_: (0, 0, 0)),
            ],
            out_specs=pl.BlockSpec((1, 1), lambda i, *_: (0, 0)),
        ),
        out_shape=jax.ShapeDtypeStruct((1, 1), jnp.float32),
    )(chunk_list, counts, pos_row, arow, brow, pos_col3, oht3)
    return jnp.reshape(out, ()) - 0.5 * jnp.reshape(dcorr, ())


# speedup vs baseline: 1.0713x; 1.0713x over previous
import functools

import numpy as np
import jax
import jax.numpy as jnp
from jax import lax
from jax.experimental import pallas as pl
from jax.experimental.pallas import tpu as pltpu
from jax.experimental.pallas import tpu_sc as plsc

BOX = 10.0
ALPHA = 2.8
R_ONSET = 1.7
R_CUTOFF = 2.0
NSIDE = 5
NCELLS = NSIDE ** 3


def _cell_adjacency() -> np.ndarray:
    ids = np.arange(NCELLS)
    x, rem = divmod(ids, NSIDE * NSIDE)
    y, z = divmod(rem, NSIDE)
    def near(a, b):
        d = np.abs(a[:, None] - b[None, :])
        return np.minimum(d, NSIDE - d) <= 1
    adj = near(x, x) & near(y, y) & near(z, z)
    return adj.astype(np.float32)


_ADJ = _cell_adjacency()

_SC_CORES = 2
_SC_SUBCORES = 16


def _sc_permute_rows(table, idx):
    n, d = table.shape
    nw = _SC_CORES * _SC_SUBCORES
    b_per_w = n // nw
    mesh = plsc.VectorSubcoreMesh(core_axis_name="c", subcore_axis_name="s")

    @functools.partial(
        pl.kernel,
        out_type=jax.ShapeDtypeStruct((n, d), jnp.float32),
        mesh=mesh,
        compiler_params=pltpu.CompilerParams(use_tc_tiling_on_sc=False),
        scratch_types=[
            pltpu.VMEM((b_per_w,), jnp.int32),
            pltpu.VMEM((b_per_w, d), jnp.float32),
            pltpu.SemaphoreType.DMA,
        ],
    )
    def gather_kernel(table_hbm, idx_hbm, out_hbm, idx_v, rows_v, sem):
        wid = lax.axis_index("s") * _SC_CORES + lax.axis_index("c")
        base = wid * b_per_w
        pltpu.sync_copy(idx_hbm.at[pl.ds(base, b_per_w)], idx_v)
        pltpu.async_copy(table_hbm.at[idx_v], rows_v, sem).wait()
        pltpu.sync_copy(rows_v, out_hbm.at[pl.ds(base, b_per_w)])

    return gather_kernel(table, idx)


def _prologue_kernel(ct_ref, ctt_ref, cad_ref, rrow_ref, rcol_ref,
                     arow_ref, brow_ref, oht_ref, dcorr_ref):
    sig8 = rcol_ref[...] + rrow_ref[...]
    eps8 = cad_ref[...]
    e_sig = jnp.exp(ALPHA * sig8)
    a8 = eps8 * e_sig * e_sig
    b8 = 2.0 * eps8 * e_sig

    ct = ct_ref[...]
    n = ct.shape[0]
    mx = jnp.max(ct, axis=1, keepdims=True)
    iota = jax.lax.broadcasted_iota(jnp.int32, (n, 8), 1)
    idx = jnp.min(jnp.where(ct == mx, iota, 8), axis=1, keepdims=True)
    idx = jnp.where(jnp.sum(ct, axis=1, keepdims=True) > 0.0, idx, 0)
    oh = (iota == idx).astype(jnp.float32)
    arow_ref[...] = jnp.dot(oh, a8, preferred_element_type=jnp.float32)
    brow_ref[...] = jnp.dot(oh, b8, preferred_element_type=jnp.float32)

    ctt = ctt_ref[...]
    mxt = jnp.max(ctt, axis=0, keepdims=True)
    iota_t = jax.lax.broadcasted_iota(jnp.int32, (8, n), 0)
    idx_t = jnp.min(jnp.where(ctt == mxt, iota_t, 8), axis=0, keepdims=True)
    idx_t = jnp.where(jnp.sum(ctt, axis=0, keepdims=True) > 0.0, idx_t, 0)
    oht_ref[...] = (iota_t == idx_t).astype(jnp.float32)

    t1 = float(np.exp(-ALPHA))
    diag_iota = jax.lax.broadcasted_iota(jnp.int32, (8, 8), 0)
    eye8 = (diag_iota == jax.lax.broadcasted_iota(jnp.int32, (8, 8), 1))
    a_d = jnp.sum(jnp.where(eye8, a8, 0.0), axis=1, keepdims=True)
    b_d = jnp.sum(jnp.where(eye8, b8, 0.0), axis=1, keepdims=True)
    av = jnp.dot(oh, a_d, preferred_element_type=jnp.float32)
    bv = jnp.dot(oh, b_d, preferred_element_type=jnp.float32)
    dcorr_ref[...] = jnp.sum((av * t1 - bv) * t1).reshape(1, 1)


def _tile_kernel(chunks_ref, counts_ref, pr_ref, arow_ref, brow_ref,
                 pc_ref, oht_ref, out_ref, *, tr, cc):
    i = pl.program_id(0)

    pr = pr_ref[...]
    arow = arow_ref[...]
    brow = brow_ref[...]

    r_on2 = R_ONSET * R_ONSET
    r_c2 = R_CUTOFF * R_CUTOFF
    inv_den = 1.0 / (r_c2 - r_on2) ** 3

    def chunk_u(c):
        pc = pc_ref[c]
        oht = oht_ref[c]
        a_t = jnp.dot(arow, oht, preferred_element_type=jnp.float32)
        b_t = jnp.dot(brow, oht, preferred_element_type=jnp.float32)

        dr2 = jnp.zeros((tr, cc), jnp.float32)
        for kk in range(3):
            d = pr[:, kk:kk + 1] - pc[kk:kk + 1, :]
            d = d - BOX * jnp.round(d * (1.0 / BOX))
            dr2 = dr2 + d * d

        safe = jnp.where(dr2 > 0.0, dr2, 1.0)
        r = jnp.sqrt(safe)
        t = jnp.exp(-ALPHA * r)
        x = jnp.minimum(jnp.maximum(r * r, r_on2), r_c2)
        s = ((r_c2 - x) ** 2 * (r_c2 + 2.0 * x - 3.0 * r_on2)) * inv_den
        u = (a_t * t - b_t) * (t * s)
        return jnp.sum(u.reshape(tr // 8, 8, cc), axis=0)

    def body(k, acc):
        return acc + chunk_u(chunks_ref[i, k])

    acc = jax.lax.fori_loop(0, counts_ref[i], body,
                            jnp.zeros((8, cc), jnp.float32))
    total = 0.5 * (jnp.sum(chunk_u(i)) + 2.0 * jnp.sum(acc))

    @pl.when(i == 0)
    def _():
        out_ref[...] = jnp.zeros((1, 1), jnp.float32)

    out_ref[...] += total.reshape(1, 1)


def kernel(positions, celltype, cadherin, radius):
    n = positions.shape[0]
    tr, cc = 128, 128
    nr, nk = n // tr, n // cc

    ci = jnp.floor(positions * (NSIDE / BOX)).astype(jnp.int32)
    ci = jnp.clip(ci, 0, NSIDE - 1)
    cid = (ci[:, 0] * NSIDE + ci[:, 1]) * NSIDE + ci[:, 2]
    perm = jnp.argsort(cid).astype(jnp.int32)
    table = jnp.concatenate(
        [positions, celltype, cid[:, None].astype(jnp.float32),
         jnp.zeros((n, 4), jnp.float32)], axis=1)
    table_s = _sc_permute_rows(table, perm)
    pos_s = table_s[:, 0:3]
    ct_s = table_s[:, 3:11]
    cid_s = table_s[:, 11].astype(jnp.int32)

    oh_cell = (cid_s[:, None] == jnp.arange(NCELLS)[None, :])
    pres_r = jnp.any(oh_cell.reshape(nr, tr, NCELLS), axis=1)
    pres_c = jnp.any(oh_cell.reshape(nk, cc, NCELLS), axis=1)
    adj = jnp.asarray(_ADJ)
    reach = pres_r.astype(jnp.float32) @ adj
    active = (reach @ pres_c.T.astype(jnp.float32)) > 0.0
    rows = jnp.broadcast_to(jnp.arange(nr)[:, None], (nr, nk))
    cols = jnp.broadcast_to(jnp.arange(nk)[None, :], (nr, nk))
    upper = active & (cols > rows)
    counts = jnp.sum(upper, axis=1, dtype=jnp.int32)
    slot = jnp.cumsum(upper, axis=1) - 1
    chunk_list = jnp.zeros((nr, nk), jnp.int32).at[
        rows.ravel(),
        jnp.where(upper, slot, nk).ravel()
    ].set(cols.astype(jnp.int32).ravel(), mode='drop')

    pos_row = table_s[:, 0:8]
    pos_col3 = jnp.transpose(pos_row.T.reshape(8, nk, cc), (1, 0, 2))
    cad8 = jnp.reshape(cadherin, (8, 8))
    cad8 = 0.5 * (cad8 + cad8.T)
    r8_row = jnp.reshape(radius[:8, 0], (1, 8))
    r8_col = jnp.reshape(radius[:8, 0], (8, 1))

    arow, brow, oht, dcorr = pl.pallas_call(
        _prologue_kernel,
        out_shape=[
            jax.ShapeDtypeStruct((n, 8), jnp.float32),
            jax.ShapeDtypeStruct((n, 8), jnp.float32),
            jax.ShapeDtypeStruct((8, n), jnp.float32),
            jax.ShapeDtypeStruct((1, 1), jnp.float32),
        ],
    )(ct_s, ct_s.T, cad8, r8_row, r8_col)
    oht3 = jnp.transpose(oht.reshape(8, nk, cc), (1, 0, 2))

    out = pl.pallas_call(
        functools.partial(_tile_kernel, tr=tr, cc=cc),
        grid_spec=pltpu.PrefetchScalarGridSpec(
            num_scalar_prefetch=2,
            grid=(nr,),
            in_specs=[
                pl.BlockSpec((tr, 8), lambda i, *_: (i, 0)),
                pl.BlockSpec((tr, 8), lambda i, *_: (i, 0)),
                pl.BlockSpec((tr, 8), lambda i, *_: (i, 0)),
                pl.BlockSpec((nk, 8, cc), lambda i, *_: (0, 0, 0)),
                pl.BlockSpec((nk, 8, cc), lambda i, *_: (0, 0, 0)),
            ],
            out_specs=pl.BlockSpec((1, 1), lambda i, *_: (0, 0)),
        ),
        out_shape=jax.ShapeDtypeStruct((1, 1), jnp.float32),
    )(chunk_list, counts, pos_row, arow, brow, pos_col3, oht3)
    return jnp.reshape(out, ()) - 0.5 * jnp.reshape(dcorr, ())
